# baseline (speedup 1.0000x reference)
import functools

import jax
import jax.numpy as jnp
from jax import lax
from jax.experimental import pallas as pl
from jax.experimental.pallas import tpu as pltpu
from jax.experimental.pallas import tpu_sc as plsc

VOCAB = 1000000
EMBED = 64
BATCH = 16384
NEG = 10

_NC = 2
_NS = 16
_NW = _NC * _NS
_LANES = 16

_BPW = BATCH // _NW
_CH = 128
_NCHUNK = _BPW // _CH


def _sc_scores():
    mesh = plsc.VectorSubcoreMesh(core_axis_name="c", subcore_axis_name="s")

    @functools.partial(
        pl.kernel,
        mesh=mesh,
        compiler_params=pltpu.CompilerParams(needs_layout_passes=False,
                                             use_tc_tiling_on_sc=False),
        out_type=jax.ShapeDtypeStruct(((1 + NEG) * BATCH,), jnp.float32),
        scratch_types=[
            pltpu.VMEM((_CH,), jnp.int32),
            pltpu.VMEM((_CH,), jnp.int32),
            pltpu.VMEM((NEG * _CH,), jnp.int32),
            pltpu.VMEM((_CH, EMBED), jnp.float32),
            pltpu.VMEM((_CH, EMBED), jnp.float32),
            pltpu.VMEM((NEG * _CH, EMBED), jnp.float32),
            pltpu.VMEM(((1 + NEG) * _CH,), jnp.float32),
            pltpu.SemaphoreType.DMA,
        ],
    )
    def sc_scores(iw_hbm, cw_hbm, neg_hbm, win_hbm, wctx_hbm, out_hbm,
                  iidx, cidx, nidx, irows, crows, nrows, scores, sem):
        wid = lax.axis_index("s") * _NC + lax.axis_index("c")
        lane0 = lax.iota(jnp.int32, _LANES) == 0

        def chunk_body(j, _):
            base = wid * _BPW + j * _CH
            pltpu.sync_copy(iw_hbm.at[pl.ds(base, _CH)], iidx)
            pltpu.sync_copy(cw_hbm.at[pl.ds(base, _CH)], cidx)
            pltpu.sync_copy(neg_hbm.at[pl.ds(base * NEG, NEG * _CH)], nidx)
            copies = [
                pltpu.async_copy(win_hbm.at[iidx], irows, sem),
                pltpu.async_copy(wctx_hbm.at[cidx], crows, sem),
            ]
            for k in range(NEG):
                copies.append(
                    pltpu.async_copy(wctx_hbm.at[nidx.at[pl.ds(k * _CH, _CH)]],
                                     nrows.at[pl.ds(k * _CH, _CH)], sem))
            for c in copies:
                c.wait()

            def elem_body(b, _):
                vin = [irows[b, pl.ds(q * _LANES, _LANES)] for q in range(4)]
                vctx = [crows[b, pl.ds(q * _LANES, _LANES)] for q in range(4)]
                acc = vin[0] * vctx[0]
                for q in range(1, 4):
                    acc = acc + vin[q] * vctx[q]
                s = jnp.sum(acc)
                plsc.store_scatter(scores, [jnp.full((_LANES,), b, jnp.int32)],
                                   jnp.full((_LANES,), s, jnp.float32),
                                   mask=lane0)
                for k in range(NEG):
                    vng = [nrows[k * _CH + b, pl.ds(q * _LANES, _LANES)]
                           for q in range(4)]
                    nacc = vin[0] * vng[0]
                    for q in range(1, 4):
                        nacc = nacc + vin[q] * vng[q]
                    ns = -jnp.sum(nacc)
                    plsc.store_scatter(
                        scores,
                        [jnp.full((_LANES,), (1 + k) * _CH + b, jnp.int32)],
                        jnp.full((_LANES,), ns, jnp.float32),
                        mask=lane0)
                return 0

            lax.fori_loop(0, _CH, elem_body, 0)
            for k in range(1 + NEG):
                pltpu.sync_copy(scores.at[pl.ds(k * _CH, _CH)],
                                out_hbm.at[pl.ds(k * BATCH + base, _CH)])
            return 0

        lax.fori_loop(0, _NCHUNK, chunk_body, 0)

    return sc_scores


_SC_SCORES = _sc_scores()

_ROWS = (1 + NEG) * BATCH // 128


def _loss_body(s_ref, o_ref):
    x = s_ref[...]
    ls = jnp.minimum(x, 0.0) - jnp.log1p(jnp.exp(-jnp.abs(x)))
    o_ref[0, 0] = -jnp.sum(ls) / BATCH


def kernel(input_word, context_word, W_in, W_ctx):
    batch_size = context_word.shape[0]
    neg_key = jax.random.key(1234)
    negative_example = jax.random.randint(neg_key, (batch_size, NEG), 0, VOCAB)
    neg_cm = (negative_example.astype(jnp.int32)
              .reshape(batch_size // _CH, _CH, NEG)
              .transpose(0, 2, 1)
              .reshape(-1))

    scores = _SC_SCORES(input_word.astype(jnp.int32),
                        context_word.astype(jnp.int32),
                        neg_cm, W_in, W_ctx)

    loss = pl.pallas_call(
        _loss_body,
        out_shape=jax.ShapeDtypeStruct((1, 1), jnp.float32),
        out_specs=pl.BlockSpec(memory_space=pltpu.SMEM),
    )(scores.reshape(_ROWS, 128))
    return loss[0, 0]


# speedup vs baseline: 1.4103x; 1.4103x over previous
import functools

import jax
import jax.numpy as jnp
from jax import lax
from jax.experimental import pallas as pl
from jax.experimental.pallas import tpu as pltpu
from jax.experimental.pallas import tpu_sc as plsc

VOCAB = 1000000
EMBED = 64
BATCH = 16384
NEG = 10

_NC = 2
_NS = 16
_NW = _NC * _NS
_LANES = 16

_BPW = BATCH // _NW
_CH = 64
_NCHUNK = _BPW // _CH
_NGRP = _CH // _LANES


def _sc_scores():
    mesh = plsc.VectorSubcoreMesh(core_axis_name="c", subcore_axis_name="s")

    @functools.partial(
        pl.kernel,
        mesh=mesh,
        compiler_params=pltpu.CompilerParams(needs_layout_passes=False,
                                             use_tc_tiling_on_sc=True),
        out_type=jax.ShapeDtypeStruct(((1 + NEG) * BATCH,), jnp.float32),
        scratch_types=[
            pltpu.VMEM((_CH,), jnp.int32),
            pltpu.VMEM((_CH,), jnp.int32),
            pltpu.VMEM((NEG * _CH,), jnp.int32),
            pltpu.VMEM((_CH, EMBED), jnp.float32),
            pltpu.VMEM((_CH, EMBED), jnp.float32),
            pltpu.VMEM((NEG * _CH, EMBED), jnp.float32),
            pltpu.VMEM(((1 + NEG) * _CH,), jnp.float32),
            pltpu.SemaphoreType.DMA,
        ],
    )
    def sc_scores(iw_hbm, cw_hbm, neg_hbm, win_hbm, wctx_hbm, out_hbm,
                  iidx, cidx, nidx, irows, crows, nrows, scores, sem):
        wid = lax.axis_index("s") * _NC + lax.axis_index("c")
        lane0 = lax.iota(jnp.int32, _LANES) == 0

        def chunk_body(j, _):
            base = wid * _BPW + j * _CH
            pltpu.sync_copy(iw_hbm.at[pl.ds(base, _CH)], iidx)
            pltpu.sync_copy(cw_hbm.at[pl.ds(base, _CH)], cidx)
            pltpu.sync_copy(neg_hbm.at[pl.ds(base * NEG, NEG * _CH)], nidx)

            def fire_body(g, _):
                goff = g * _LANES
                iv = iidx[pl.ds(goff, _LANES)]
                cv = cidx[pl.ds(goff, _LANES)]
                for j16 in range(_LANES):
                    l = goff + j16
                    si = lax.index_in_dim(iv, j16, 0, keepdims=False)
                    pltpu.async_copy(win_hbm.at[pl.ds(si, 1), :],
                                     irows.at[pl.ds(l, 1), :], sem)
                    sc = lax.index_in_dim(cv, j16, 0, keepdims=False)
                    pltpu.async_copy(wctx_hbm.at[pl.ds(sc, 1), :],
                                     crows.at[pl.ds(l, 1), :], sem)
                for k in range(NEG):
                    nv = nidx[pl.ds(k * _CH + goff, _LANES)]
                    for j16 in range(_LANES):
                        sn = lax.index_in_dim(nv, j16, 0, keepdims=False)
                        pltpu.async_copy(
                            wctx_hbm.at[pl.ds(sn, 1), :],
                            nrows.at[pl.ds(k * _CH + goff + j16, 1), :], sem)
                return 0

            lax.fori_loop(0, _NGRP, fire_body, 0)
            pltpu.make_async_copy(win_hbm.at[pl.ds(0, _CH), :], irows,
                                  sem).wait()
            pltpu.make_async_copy(win_hbm.at[pl.ds(0, _CH), :], crows,
                                  sem).wait()
            pltpu.make_async_copy(win_hbm.at[pl.ds(0, NEG * _CH), :], nrows,
                                  sem).wait()

            def elem_body(b, _):
                vin = [irows[b, pl.ds(q * _LANES, _LANES)] for q in range(4)]
                vctx = [crows[b, pl.ds(q * _LANES, _LANES)] for q in range(4)]
                acc = vin[0] * vctx[0]
                for q in range(1, 4):
                    acc = acc + vin[q] * vctx[q]
                s = jnp.sum(acc)
                plsc.store_scatter(scores, [jnp.full((_LANES,), b, jnp.int32)],
                                   jnp.full((_LANES,), s, jnp.float32),
                                   mask=lane0)
                for k in range(NEG):
                    vng = [nrows[k * _CH + b, pl.ds(q * _LANES, _LANES)]
                           for q in range(4)]
                    nacc = vin[0] * vng[0]
                    for q in range(1, 4):
                        nacc = nacc + vin[q] * vng[q]
                    ns = -jnp.sum(nacc)
                    plsc.store_scatter(
                        scores,
                        [jnp.full((_LANES,), (1 + k) * _CH + b, jnp.int32)],
                        jnp.full((_LANES,), ns, jnp.float32),
                        mask=lane0)
                return 0

            lax.fori_loop(0, _CH, elem_body, 0)
            for k in range(1 + NEG):
                pltpu.sync_copy(scores.at[pl.ds(k * _CH, _CH)],
                                out_hbm.at[pl.ds(k * BATCH + base, _CH)])
            return 0

        lax.fori_loop(0, _NCHUNK, chunk_body, 0)

    return sc_scores


_SC_SCORES = _sc_scores()

_ROWS = (1 + NEG) * BATCH // 128


def _loss_body(s_ref, o_ref):
    x = s_ref[...]
    ls = jnp.minimum(x, 0.0) - jnp.log1p(jnp.exp(-jnp.abs(x)))
    o_ref[0, 0] = -jnp.sum(ls) / BATCH


def kernel(input_word, context_word, W_in, W_ctx):
    batch_size = context_word.shape[0]
    neg_key = jax.random.key(1234)
    negative_example = jax.random.randint(neg_key, (batch_size, NEG), 0, VOCAB)
    neg_cm = (negative_example.astype(jnp.int32)
              .reshape(batch_size // _CH, _CH, NEG)
              .transpose(0, 2, 1)
              .reshape(-1))

    scores = _SC_SCORES(input_word.astype(jnp.int32),
                        context_word.astype(jnp.int32),
                        neg_cm, W_in, W_ctx)

    loss = pl.pallas_call(
        _loss_body,
        out_shape=jax.ShapeDtypeStruct((1, 1), jnp.float32),
        out_specs=pl.BlockSpec(memory_space=pltpu.SMEM),
    )(scores.reshape(_ROWS, 128))
    return loss[0, 0]
